# baseline (speedup 1.0000x reference)
import functools

import jax
import jax.numpy as jnp
from jax import lax
from jax.experimental import pallas as pl
from jax.experimental.pallas import tpu as pltpu
from jax.experimental.pallas import tpu_sc as plsc

D = 64
SEG = 10
CLK = 20
SL = 20
LL = 50
B = 1024

NC = 2
NS = 16
NW = NC * NS
QPT = B // NW
CH = 32
SLP = 32
LLP = 64
SEGP = 16
CLKP = 32
WSUB = 4
ISUB = 8


def _f32(x):
    return x.astype(jnp.float32)


_MUL = {SEG: 6554, CLK: 3277, SL: 3277, LL: 1311}


def _mask_pass(flat_ref, lrep_ref, slots, n, vmax):
    lane = lax.iota(jnp.int32, 16)
    m = _MUL[slots]
    for j in range(n * slots // 16):
        flat = lane + (16 * j)
        ent = lax.shift_right_logical(flat * m, 16)
        slot = flat - ent * slots
        v = flat_ref[pl.ds(16 * j, 16)]
        v = jnp.minimum(jnp.maximum(v, 0), vmax)
        ln = lrep_ref[pl.ds(16 * j, 16)]
        flat_ref[pl.ds(16 * j, 16)] = jnp.where(slot < ln, v, 0)


def _sc_a(
    qid_hbm, qseg_hbm, qseglen_hbm, qclick_hbm, qclicklen_hbm,
    lhis_hbm, shis_hbm, llen_hbm, slen_hbm,
    seg_s_o, clk_s_o, seg_l_o, clk_l_o, seg_q_o,
    slen_s_o, clen_s_o, slen_l_o, clen_l_o, slen_q_o,
    slsel_o, llsel_o,
    qid_v, shv, lhv, slv, llv,
    pseg_s, pclk_s, psl_s, pcl_s, pseg_l, pclk_l, psl_l, pcl_l,
    seg_q_v, sl_q_v, gsem, wsem,
):
    wid = lax.axis_index("s") * NC + lax.axis_index("c")
    qb = wid * QPT

    pltpu.sync_copy(qid_hbm.at[pl.ds(qb, QPT)], qid_v)
    pltpu.sync_copy(shis_hbm.at[qid_v], shv)
    pltpu.sync_copy(lhis_hbm.at[qid_v], lhv)
    pltpu.sync_copy(slen_hbm.at[qid_v], slv)
    pltpu.sync_copy(llen_hbm.at[qid_v], llv)

    def qbody(q, _):
        i_s = shv.at[q]
        i_l = lhv.at[q]
        hs = [
            pltpu.async_copy(qseg_hbm.at[i_s], pseg_s, gsem),
            pltpu.async_copy(qclick_hbm.at[i_s], pclk_s, gsem),
            pltpu.async_copy(qseglen_hbm.at[i_s], psl_s, gsem),
            pltpu.async_copy(qclicklen_hbm.at[i_s], pcl_s, gsem),
            pltpu.async_copy(qseg_hbm.at[i_l], pseg_l, gsem),
            pltpu.async_copy(qclick_hbm.at[i_l], pclk_l, gsem),
            pltpu.async_copy(qseglen_hbm.at[i_l], psl_l, gsem),
            pltpu.async_copy(qclicklen_hbm.at[i_l], pcl_l, gsem),
        ]
        for h in hs:
            h.wait()
        rs = (qb + q) * SLP
        rl = (qb + q) * LLP
        hs = [
            pltpu.async_copy(pseg_s, seg_s_o.at[pl.ds(rs, SLP)], wsem),
            pltpu.async_copy(pclk_s, clk_s_o.at[pl.ds(rs, SLP)], wsem),
            pltpu.async_copy(psl_s, slen_s_o.at[pl.ds(rs, SLP)], wsem),
            pltpu.async_copy(pcl_s, clen_s_o.at[pl.ds(rs, SLP)], wsem),
            pltpu.async_copy(pseg_l, seg_l_o.at[pl.ds(rl, LLP)], wsem),
            pltpu.async_copy(pclk_l, clk_l_o.at[pl.ds(rl, LLP)], wsem),
            pltpu.async_copy(psl_l, slen_l_o.at[pl.ds(rl, LLP)], wsem),
            pltpu.async_copy(pcl_l, clen_l_o.at[pl.ds(rl, LLP)], wsem),
        ]
        for h in hs:
            h.wait()
        return 0

    lax.fori_loop(0, QPT, qbody, 0)
    pltpu.sync_copy(qseg_hbm.at[qid_v], seg_q_v)
    pltpu.sync_copy(qseglen_hbm.at[qid_v], sl_q_v)
    pltpu.sync_copy(seg_q_v, seg_q_o.at[pl.ds(qb, QPT)])
    pltpu.sync_copy(sl_q_v, slen_q_o.at[pl.ds(qb, QPT)])
    pltpu.sync_copy(slv, slsel_o.at[pl.ds(qb, QPT)])
    pltpu.sync_copy(llv, llsel_o.at[pl.ds(qb, QPT)])


def _sc_b(
    seg_s_f, srep_s, clk_s_f, crep_s,
    seg_l_f, srep_l, clk_l_f, crep_l,
    seg_q_f, srep_q,
    wemb_hbm, iemb_hbm,
    qsum_s, isum_s, qsum_l, isum_l, qsum_q,
    segflat, slrep, clkflat, clrep, wrows, crows, qstage, istage,
    in_sem, gat_sem,
):
    wid = lax.axis_index("s") * NC + lax.axis_index("c")

    def chunk_body(seg_f, srep, clk_f, crep, ebase, want_items, qout, iout):
        hs = [
            pltpu.async_copy(seg_f.at[pl.ds(ebase * SEG, CH * SEG)],
                             segflat, in_sem),
            pltpu.async_copy(srep.at[pl.ds(ebase * SEG, CH * SEG)],
                             slrep, in_sem),
        ]
        if want_items:
            hs += [
                pltpu.async_copy(clk_f.at[pl.ds(ebase * CLK, CH * CLK)],
                                 clkflat, in_sem),
                pltpu.async_copy(crep.at[pl.ds(ebase * CLK, CH * CLK)],
                                 clrep, in_sem),
            ]
        for h in hs:
            h.wait()
        _mask_pass(segflat, slrep, SEG, CH, 99999)
        if want_items:
            _mask_pass(clkflat, clrep, CLK, CH, 999999)
        hs = []
        nw = CH * SEG // WSUB
        for k in range(WSUB):
            hs.append(pltpu.async_copy(
                wemb_hbm.at[segflat.at[pl.ds(k * nw, nw)]],
                wrows.at[pl.ds(k * nw, nw)], gat_sem))
        if want_items:
            ni = CH * CLK // ISUB
            for k in range(ISUB):
                hs.append(pltpu.async_copy(
                    iemb_hbm.at[clkflat.at[pl.ds(k * ni, ni)]],
                    crows.at[pl.ds(k * ni, ni)], gat_sem))
        for h in hs:
            h.wait()

        def pool_body(e, _):
            for j in range(4):
                acc = wrows[e * SEG, pl.ds(16 * j, 16)]
                for s in range(1, SEG):
                    acc = acc + wrows[e * SEG + s, pl.ds(16 * j, 16)]
                qstage[e, pl.ds(16 * j, 16)] = acc
            if want_items:
                for j in range(4):
                    acc = crows[e * CLK, pl.ds(16 * j, 16)]
                    for s in range(1, CLK):
                        acc = acc + crows[e * CLK + s, pl.ds(16 * j, 16)]
                    istage[e, pl.ds(16 * j, 16)] = acc
            return 0

        lax.fori_loop(0, CH, pool_body, 0)
        pltpu.sync_copy(qstage, qout.at[pl.ds(ebase, CH)])
        if want_items:
            pltpu.sync_copy(istage, iout.at[pl.ds(ebase, CH)])

    def region(seg_f, srep, clk_f, crep, tile_base, nchunks, qout, iout):
        def body(c, _):
            chunk_body(seg_f, srep, clk_f, crep, tile_base + c * CH,
                       True, qout, iout)
            return 0
        lax.fori_loop(0, nchunks, body, 0)

    region(seg_s_f, srep_s, clk_s_f, crep_s, wid * QPT * SL,
           QPT * SL // CH, qsum_s, isum_s)
    region(seg_l_f, srep_l, clk_l_f, crep_l, wid * QPT * LL,
           QPT * LL // CH, qsum_l, isum_l)
    chunk_body(seg_q_f, srep_q, None, None, wid * QPT, False, qsum_q, None)


_SC_PARAMS = pltpu.CompilerParams(use_tc_tiling_on_sc=False)


def _run_sc_stage(query_ids, query_seg, query_seg_lens, query_click,
                  query_click_lens, query_long_his, query_short_his,
                  long_lens, short_lens, word_emb, item_emb):
    i32 = jnp.int32
    f32 = jnp.float32
    mesh = plsc.VectorSubcoreMesh(core_axis_name="c", subcore_axis_name="s")

    a_out = (
        jax.ShapeDtypeStruct((B * SLP, SEGP), i32),
        jax.ShapeDtypeStruct((B * SLP, CLKP), i32),
        jax.ShapeDtypeStruct((B * LLP, SEGP), i32),
        jax.ShapeDtypeStruct((B * LLP, CLKP), i32),
        jax.ShapeDtypeStruct((B, SEGP), i32),
        jax.ShapeDtypeStruct((B * SLP,), i32),
        jax.ShapeDtypeStruct((B * SLP,), i32),
        jax.ShapeDtypeStruct((B * LLP,), i32),
        jax.ShapeDtypeStruct((B * LLP,), i32),
        jax.ShapeDtypeStruct((B,), i32),
        jax.ShapeDtypeStruct((B,), i32),
        jax.ShapeDtypeStruct((B,), i32),
    )
    a_scratch = [
        pltpu.VMEM((QPT,), i32),
        pltpu.VMEM((QPT, SLP), i32),
        pltpu.VMEM((QPT, LLP), i32),
        pltpu.VMEM((QPT,), i32),
        pltpu.VMEM((QPT,), i32),
        pltpu.VMEM((SLP, SEGP), i32),
        pltpu.VMEM((SLP, CLKP), i32),
        pltpu.VMEM((SLP,), i32),
        pltpu.VMEM((SLP,), i32),
        pltpu.VMEM((LLP, SEGP), i32),
        pltpu.VMEM((LLP, CLKP), i32),
        pltpu.VMEM((LLP,), i32),
        pltpu.VMEM((LLP,), i32),
        pltpu.VMEM((QPT, SEGP), i32),
        pltpu.VMEM((QPT,), i32),
        pltpu.SemaphoreType.DMA,
        pltpu.SemaphoreType.DMA,
    ]
    (seg_s_p, clk_s_p, seg_l_p, clk_l_p, seg_q, slen_sp, clen_sp, slen_lp,
     clen_lp, slen_q, slsel, llsel) = pl.kernel(
        _sc_a, out_type=a_out, mesh=mesh, scratch_types=a_scratch,
        compiler_params=_SC_PARAMS,
    )(query_ids,
      jnp.pad(query_seg, ((0, 0), (0, SEGP - SEG))),
      query_seg_lens,
      jnp.pad(query_click, ((0, 0), (0, CLKP - CLK))),
      query_click_lens,
      jnp.pad(query_long_his, ((0, 0), (0, LLP - LL))),
      jnp.pad(query_short_his, ((0, 0), (0, SLP - SL))),
      long_lens, short_lens)
    slen_s = slen_sp.reshape(B, SLP)[:, :SL].reshape(-1)
    clen_s = clen_sp.reshape(B, SLP)[:, :SL].reshape(-1)
    slen_l = slen_lp.reshape(B, LLP)[:, :LL].reshape(-1)
    clen_l = clen_lp.reshape(B, LLP)[:, :LL].reshape(-1)
    seg_s = seg_s_p.reshape(B, SLP, SEGP)[:, :SL, :SEG]
    clk_s = clk_s_p.reshape(B, SLP, CLKP)[:, :SL, :CLK]
    seg_l = seg_l_p.reshape(B, LLP, SEGP)[:, :LL, :SEG]
    clk_l = clk_l_p.reshape(B, LLP, CLKP)[:, :LL, :CLK]
    seg_q = seg_q[:, :SEG]

    b_out = (
        jax.ShapeDtypeStruct((B * SL, D), f32),
        jax.ShapeDtypeStruct((B * SL, D), f32),
        jax.ShapeDtypeStruct((B * LL, D), f32),
        jax.ShapeDtypeStruct((B * LL, D), f32),
        jax.ShapeDtypeStruct((B, D), f32),
    )
    b_scratch = [
        pltpu.VMEM((CH * SEG,), i32),
        pltpu.VMEM((CH * SEG,), i32),
        pltpu.VMEM((CH * CLK,), i32),
        pltpu.VMEM((CH * CLK,), i32),
        pltpu.VMEM((CH * SEG, D), f32),
        pltpu.VMEM((CH * CLK, D), f32),
        pltpu.VMEM((CH, D), f32),
        pltpu.VMEM((CH, D), f32),
        pltpu.SemaphoreType.DMA,
        pltpu.SemaphoreType.DMA,
    ]
    qsum_s, isum_s, qsum_l, isum_l, qsum_q = pl.kernel(
        _sc_b, out_type=b_out, mesh=mesh, scratch_types=b_scratch,
        compiler_params=_SC_PARAMS,
    )(seg_s.reshape(-1), jnp.repeat(slen_s, SEG),
      clk_s.reshape(-1), jnp.repeat(clen_s, CLK),
      seg_l.reshape(-1), jnp.repeat(slen_l, SEG),
      clk_l.reshape(-1), jnp.repeat(clen_l, CLK),
      seg_q.reshape(-1), jnp.repeat(slen_q, SEG),
      word_emb, item_emb)

    return (qsum_s, isum_s, slen_s, clen_s, qsum_l, isum_l, slen_l, clen_l,
            qsum_q, slen_q, slsel, llsel)


NB = 64


def _tc_stage(qs_s, is_s, qs_l, is_l, qs_q,
              slen_s, clen_s, slen_l, clen_l, slen_q, slsel, llsel,
              w0, i0, W_qp, b_qp, W_a1, b_a1, W_a2, b_a2,
              Wih_s, Whh_s, bih_s, bhh_s, Wih_l, Whh_l, bih_l, bhh_l,
              W_sp, W_lp, out_ref, x_s, x_l, hst):
    f32 = jnp.float32

    def pooled(qsum, slen, slots, row0):
        lf = _f32(slen)
        return (qsum - (slots - lf) * row0) / lf

    def qproj(x):
        return jnp.tanh(
            jnp.dot(x, W_qp[...].T, preferred_element_type=f32) + b_qp[...])

    qe_s = qproj(pooled(qs_s[...].reshape(NB * SL, D),
                        slen_s[...], SEG, w0[...]))
    x_s[:, :, 0:D] = qe_s.reshape(NB, SL, D)
    x_s[:, :, D:2 * D] = pooled(is_s[...].reshape(NB * SL, D),
                                clen_s[...], CLK,
                                i0[...]).reshape(NB, SL, D)
    qe_l = qproj(pooled(qs_l[...].reshape(NB * LL, D),
                        slen_l[...], SEG, w0[...]))
    x_l[:, :, 0:D] = qe_l.reshape(NB, LL, D)
    x_l[:, :, D:2 * D] = pooled(is_l[...].reshape(NB * LL, D),
                                clen_l[...], CLK,
                                i0[...]).reshape(NB, LL, D)
    qemb = qproj(pooled(qs_q[...], slen_q[...], SEG, w0[...]))

    def gru(x, h, Wih, Whh, bih, bhh):
        gi = jnp.dot(x, Wih[...].T, preferred_element_type=f32) + bih[...]
        gh = jnp.dot(h, Whh[...].T, preferred_element_type=f32) + bhh[...]
        r = jax.nn.sigmoid(gi[:, :D] + gh[:, :D])
        z = jax.nn.sigmoid(gi[:, D:2 * D] + gh[:, D:2 * D])
        n = jnp.tanh(gi[:, 2 * D:] + r * gh[:, 2 * D:])
        return (1.0 - z) * n + z * h

    slsel_v = slsel[...]

    def sbody(t, carry):
        h, hsel = carry
        x = x_s[:, pl.ds(t, 1), :].reshape(NB, 2 * D)
        h2 = gru(x, h, Wih_s, Whh_s, bih_s, bhh_s)
        hsel = jnp.where(slsel_v == t + 1, h2, hsel)
        return h2, hsel

    h0 = jnp.zeros((NB, D), f32)
    _, hsel = lax.fori_loop(0, SL, sbody, (h0, h0))
    p_short = jnp.dot(hsel, W_sp[...].T, preferred_element_type=f32)

    u = jnp.dot(W_a2[...], W_a1[...], preferred_element_type=f32)
    c = jnp.sum(W_a2[...] * b_a1[...]) + b_a2[0, 0]
    qdot = jnp.sum(qemb * u[:, :D], axis=1, keepdims=True)
    uh = u[:, D:]

    def lbody(t, h):
        x = x_l[:, pl.ds(t, 1), :].reshape(NB, 2 * D)
        h2 = gru(x, h, Wih_l, Whh_l, bih_l, bhh_l)
        hst[:, pl.ds(t, 1), :] = h2.reshape(NB, 1, D)
        return h2

    lax.fori_loop(0, LL, lbody, h0)

    cols = [jnp.sum(hst[:, t, :] * uh, axis=1, keepdims=True)
            for t in range(LL)]
    e = jnp.tanh(jnp.concatenate(cols, axis=1) + qdot + c)
    tmask = lax.broadcasted_iota(jnp.int32, (NB, LL), 1) < llsel[...]
    e = jnp.where(tmask, e, -1e9)
    m = jnp.max(e, axis=1, keepdims=True)
    a = jnp.exp(e - m)
    att = a / jnp.sum(a, axis=1, keepdims=True)

    li = jnp.zeros((NB, D), f32)
    for t in range(LL):
        li = li + att[:, t:t + 1] * hst[:, t, :]
    p_long = jnp.dot(li, W_lp[...].T, preferred_element_type=f32)

    out_ref[...] = p_short + p_long + qemb


def _run_tc_stage(sc_outs, w_row0, i_row0, W_qp, b_qp, W_a1, b_a1, W_a2, b_a2,
                  Wih_s, Whh_s, bih_s, bhh_s, Wih_l, Whh_l, bih_l, bhh_l,
                  W_sp, W_lp):
    (qsum_s, isum_s, slen_s, clen_s, qsum_l, isum_l, slen_l, clen_l,
     qsum_q, slen_q, slsel, llsel) = sc_outs
    f32 = jnp.float32
    grid = (B // NB,)

    def blk(*shape):
        return pl.BlockSpec((NB,) + shape, lambda i: (i,) + (0,) * len(shape))

    def rep(arr):
        nd = arr.ndim
        return pl.BlockSpec(arr.shape, lambda i, _n=nd: (0,) * _n)

    weights = (w_row0, i_row0, W_qp, b_qp, W_a1, b_a1, W_a2, b_a2,
               Wih_s, Whh_s, bih_s, bhh_s, Wih_l, Whh_l, bih_l, bhh_l,
               W_sp, W_lp)
    lblk_s = pl.BlockSpec((NB * SL, 1), lambda i: (i, 0))
    lblk_l = pl.BlockSpec((NB * LL, 1), lambda i: (i, 0))
    in_specs = [
        blk(SL, D), blk(SL, D), blk(LL, D), blk(LL, D), blk(D),
        lblk_s, lblk_s, lblk_l, lblk_l, blk(1), blk(1), blk(1),
    ] + [rep(w) for w in weights]
    return pl.pallas_call(
        _tc_stage,
        grid=grid,
        in_specs=in_specs,
        out_specs=pl.BlockSpec((NB, D), lambda i: (i, 0)),
        out_shape=jax.ShapeDtypeStruct((B, D), f32),
        scratch_shapes=[
            pltpu.VMEM((NB, SL, 2 * D), f32),
            pltpu.VMEM((NB, LL, 2 * D), f32),
            pltpu.VMEM((NB, LL, D), f32),
        ],
    )(
        qsum_s.reshape(B, SL, D), isum_s.reshape(B, SL, D),
        qsum_l.reshape(B, LL, D), isum_l.reshape(B, LL, D), qsum_q,
        slen_s.reshape(B * SL, 1), clen_s.reshape(B * SL, 1),
        slen_l.reshape(B * LL, 1), clen_l.reshape(B * LL, 1),
        slen_q.reshape(B, 1), slsel.reshape(B, 1), llsel.reshape(B, 1),
        *weights,
    )


def kernel(query_ids, query_seg, query_seg_lens, query_click,
           query_click_lens, query_long_his, query_short_his, long_lens,
           short_lens, word_emb, item_emb, W_qp, b_qp, W_a1, b_a1, W_a2,
           b_a2, Wih_s, Whh_s, bih_s, bhh_s, Wih_l, Whh_l, bih_l, bhh_l,
           W_sp, W_lp):
    sc_outs = _run_sc_stage(
        query_ids, query_seg, query_seg_lens, query_click, query_click_lens,
        query_long_his, query_short_his, long_lens, short_lens,
        word_emb, item_emb)
    return _run_tc_stage(
        sc_outs,
        word_emb[0:1], item_emb[0:1],
        W_qp, b_qp.reshape(1, D),
        W_a1, b_a1.reshape(1, 512), W_a2, b_a2.reshape(1, 1),
        Wih_s, Whh_s, bih_s.reshape(1, 3 * D), bhh_s.reshape(1, 3 * D),
        Wih_l, Whh_l, bih_l.reshape(1, 3 * D), bhh_l.reshape(1, 3 * D),
        W_sp, W_lp)
